# Initial kernel scaffold; baseline (speedup 1.0000x reference)
import jax
import jax.numpy as jnp
from jax.experimental import pallas as pl


def kernel(x, edge_index, edge_attr, u, batch, W1, b1, W2, b2, W3, b3, W4, b4):
    raise NotImplementedError("write your pallas kernel here")


# speedup vs baseline: 1.5720x; 1.5720x over previous
import functools

import jax
import jax.numpy as jnp
from jax import lax
from jax.experimental import pallas as pl
from jax.experimental.pallas import tpu as pltpu
from jax.experimental.pallas import tpu_sc as plsc

NS = 16
NC = 2
WIN = 896
SUB = 128
BE = 8192
BN = 800


def _edge_mlp_body(ea_ref, w1_ref, b1_ref, out_ref):
    ea = ea_ref[...]
    acc = jnp.broadcast_to(b1_ref[...], (ea.shape[0], 64))
    for k in range(4):
        acc = acc + ea[:, k:k + 1] * w1_ref[k:k + 1, :]
    out_ref[...] = jnp.maximum(acc, 0.0)


def _scatter_body(n_rows_pt, n_wins, row_ref, a_ref, z2_ref, o2_ref,
                  acc_out, cnt_out, idx2, vals, acc_sp):
    core = lax.axis_index("c")
    sid = lax.axis_index("s")
    r0 = sid * n_rows_pt
    nz = n_rows_pt // WIN
    ept = n_wins * WIN

    for p in range(3):
        pltpu.sync_copy(z2_ref, vals)
        for kk in range(nz):
            pltpu.sync_copy(vals, acc_sp.at[pl.ds(r0 + kk * WIN, WIN), :])
        plsc.subcore_barrier()

        if p < 2:
            fc = core * 2 + p

            def win(w, carry):
                e0 = sid * ept + w * WIN
                g0 = sid * (ept // SUB) + w * (WIN // SUB)
                pltpu.sync_copy(row_ref.at[pl.ds(g0, WIN // SUB), :], idx2)
                pltpu.sync_copy(a_ref.at[pl.ds(e0, WIN), pl.ds(fc * 16, 16)],
                                vals)
                for j in range(WIN // SUB):
                    pltpu.sync_copy(vals.at[pl.ds(j * SUB, SUB), :],
                                    acc_sp.at[idx2.at[j]], add=True)
                return carry

            lax.fori_loop(0, n_wins, win, 0)
        else:
            pltpu.sync_copy(o2_ref, vals)
            half = sid * (ept // 2) + core * (NS * ept // 2)

            def cwin(w, carry):
                g0 = (half + w * WIN) // SUB
                pltpu.sync_copy(row_ref.at[pl.ds(g0, WIN // SUB), :], idx2)
                for j in range(WIN // SUB):
                    pltpu.sync_copy(vals.at[pl.ds(j * SUB, SUB), :],
                                    acc_sp.at[idx2.at[j]], add=True)
                return carry

            lax.fori_loop(0, n_wins // 2, cwin, 0)
        plsc.subcore_barrier()

        for kk in range(nz):
            rr = r0 + kk * WIN
            pltpu.sync_copy(acc_sp.at[pl.ds(rr, WIN), :], vals)
            if p < 2:
                pltpu.sync_copy(vals, acc_out.at[core * 2 + p,
                                                 pl.ds(rr, WIN), :])
            else:
                pltpu.sync_copy(vals, cnt_out.at[core, pl.ds(rr, WIN), :])
        plsc.subcore_barrier()


def _node_mlp_body(acc_ref, cnt_ref, w23_ref, bbf_ref, b3_ref, w4_ref, b4_ref,
                   out_ref):
    total = cnt_ref[...]
    denom = jnp.maximum(total, 1.0)
    f = jnp.where(total > 0.0, 1.0, 0.0)
    mean_a = jnp.concatenate(
        [acc_ref[0], acc_ref[1], acc_ref[2], acc_ref[3]], axis=1) / denom
    h2 = jnp.dot(mean_a, w23_ref[...], preferred_element_type=jnp.float32)
    h2 = jnp.maximum(h2 + f * bbf_ref[...] + b3_ref[...], 0.0)
    out = jnp.dot(h2, w4_ref[...], preferred_element_type=jnp.float32)
    out_ref[...] = out + b4_ref[...]


def kernel(x, edge_index, edge_attr, u, batch, W1, b1, W2, b2, W3, b3, W4, b4):
    N = x.shape[0]
    E = edge_attr.shape[0]
    f32 = jnp.float32

    n_rows_pt = -(-(-(-N // NS)) // WIN) * WIN
    Np = NS * n_rows_pt
    n_wins = (E + NS * WIN - 1) // (NS * WIN)
    Ep = NS * n_wins * WIN
    pad_e = Ep - E
    pad_rows = Np - N

    row = edge_index[0]
    pad_idx = N + (jnp.arange(pad_e, dtype=jnp.int32) % pad_rows)
    row_p = jnp.concatenate([row, pad_idx]).reshape(Ep // SUB, SUB)
    ea_p = jnp.concatenate([edge_attr, jnp.zeros((pad_e, 4), f32)])

    a = pl.pallas_call(
        _edge_mlp_body,
        grid=(Ep // BE,),
        in_specs=[
            pl.BlockSpec((BE, 4), lambda i: (i, 0)),
            pl.BlockSpec((4, 64), lambda i: (0, 0)),
            pl.BlockSpec((1, 64), lambda i: (0, 0)),
        ],
        out_specs=pl.BlockSpec((BE, 64), lambda i: (i, 0)),
        out_shape=jax.ShapeDtypeStruct((Ep, 64), f32),
    )(ea_p, W1, b1.reshape(1, 64))

    z2 = jnp.zeros((WIN, 16), f32)
    o2 = jnp.ones((WIN, 16), f32)
    mesh = plsc.VectorSubcoreMesh(core_axis_name="c", subcore_axis_name="s")
    sc_fn = pl.kernel(
        functools.partial(_scatter_body, n_rows_pt, n_wins),
        out_type=(jax.ShapeDtypeStruct((4, Np, 16), f32),
                  jax.ShapeDtypeStruct((NC, Np, 16), f32)),
        mesh=mesh,
        compiler_params=pltpu.CompilerParams(use_tc_tiling_on_sc=False),
        scratch_types=[
            pltpu.VMEM((WIN // SUB, SUB), jnp.int32),
            pltpu.VMEM((WIN, 16), f32),
            pltpu.VMEM_SHARED((Np, 16), f32),
        ],
    )
    acc, cnt = sc_fn(row_p, a, z2, o2)
    cnt_t = (cnt[0, :, 0] + cnt[1, :, 0]).reshape(Np, 1)

    W23 = W2 @ W3
    bbf = (b2 @ W3).reshape(1, 256)
    out = pl.pallas_call(
        _node_mlp_body,
        grid=(N // BN,),
        in_specs=[
            pl.BlockSpec((4, BN, 16), lambda i: (0, i, 0)),
            pl.BlockSpec((BN, 1), lambda i: (i, 0)),
            pl.BlockSpec((64, 256), lambda i: (0, 0)),
            pl.BlockSpec((1, 256), lambda i: (0, 0)),
            pl.BlockSpec((1, 256), lambda i: (0, 0)),
            pl.BlockSpec((256, 512), lambda i: (0, 0)),
            pl.BlockSpec((1, 512), lambda i: (0, 0)),
        ],
        out_specs=pl.BlockSpec((BN, 512), lambda i: (i, 0)),
        out_shape=jax.ShapeDtypeStruct((N, 512), f32),
    )(acc, cnt_t, W23, bbf, b3.reshape(1, 256), W4, b4.reshape(1, 512))
    return out
